# baseline (speedup 1.0000x reference)
import jax
import jax.numpy as jnp
from jax.experimental import pallas as pl
from jax.experimental.pallas import tpu as pltpu
from functools import partial

B, T, D = 16, 1024, 32
N = B * T
M = 8192
TILE_T = 512
CHUNK_M = 2048
N_CHUNKS = M // CHUNK_M
COMMITMENT_COST = 0.25


def _vq_kernel(x_ref, emb_ref, qste_ref, idx_ref, counts_ref, lt_ref, np_ref):
    step = pl.program_id(0)
    x = x_ref[...]
    x2 = jnp.sum(x * x, axis=1)

    big = jnp.float32(jnp.inf)
    run_min = jnp.full((TILE_T,), big, dtype=jnp.float32)
    run_arg = jnp.zeros((TILE_T,), dtype=jnp.int32)

    def dist_body(c, carry):
        run_min, run_arg = carry
        emb_c = emb_ref[pl.ds(c * CHUNK_M, CHUNK_M), :]
        e2 = jnp.sum(emb_c * emb_c, axis=1)
        mm = jax.lax.dot_general(
            x.astype(jnp.bfloat16), emb_c.astype(jnp.bfloat16),
            (((1,), (1,)), ((), ())),
            preferred_element_type=jnp.float32)
        dist = (x2[:, None] + e2[None, :]) - 2.0 * mm
        cmin = jnp.min(dist, axis=1)
        iota = jax.lax.broadcasted_iota(jnp.int32, (TILE_T, CHUNK_M), 1)
        carg = jnp.min(jnp.where(dist == cmin[:, None], iota, M), axis=1)
        better = cmin < run_min
        run_arg = jnp.where(better, carg + c * CHUNK_M, run_arg)
        run_min = jnp.where(better, cmin, run_min)
        return run_min, run_arg

    run_min, run_arg = jax.lax.fori_loop(0, N_CHUNKS, dist_body, (run_min, run_arg))
    idx_ref[...] = run_arg

    @pl.when(step == 0)
    def _():
        counts_ref[...] = jnp.zeros((M,), jnp.float32)

    q = jnp.zeros((TILE_T, D), dtype=jnp.float32)

    def gather_body(c, q):
        emb_c = emb_ref[pl.ds(c * CHUNK_M, CHUNK_M), :]
        iota = jax.lax.broadcasted_iota(jnp.int32, (TILE_T, CHUNK_M), 1)
        onehot = (run_arg[:, None] == iota + c * CHUNK_M).astype(jnp.float32)
        q = q + jax.lax.dot_general(
            onehot, emb_c, (((1,), (0,)), ((), ())),
            preferred_element_type=jnp.float32,
            precision=jax.lax.Precision.HIGHEST)
        counts_ref[pl.ds(c * CHUNK_M, CHUNK_M)] += jnp.sum(onehot, axis=0)
        return q

    q = jax.lax.fori_loop(0, N_CHUNKS, gather_body, q)

    qste_ref[...] = x + (q - x)
    diff = (x - q)
    lt = jnp.mean(diff * diff, axis=1)
    npad = (jnp.sum(jnp.abs(x), axis=1) > 0).astype(jnp.float32)
    lt_ref[...] = lt * npad
    np_ref[...] = npad


@jax.jit
def kernel(x, embedding):
    x_flat = x.reshape(N, D)
    grid = (N // TILE_T,)
    qste, idx, counts, lt, npad = pl.pallas_call(
        _vq_kernel,
        grid=grid,
        in_specs=[
            pl.BlockSpec((TILE_T, D), lambda i: (i, 0)),
            pl.BlockSpec((M, D), lambda i: (0, 0)),
        ],
        out_specs=[
            pl.BlockSpec((TILE_T, D), lambda i: (i, 0)),
            pl.BlockSpec((TILE_T,), lambda i: (i,)),
            pl.BlockSpec((M,), lambda i: (0,)),
            pl.BlockSpec((TILE_T,), lambda i: (i,)),
            pl.BlockSpec((TILE_T,), lambda i: (i,)),
        ],
        out_shape=[
            jax.ShapeDtypeStruct((N, D), jnp.float32),
            jax.ShapeDtypeStruct((N,), jnp.int32),
            jax.ShapeDtypeStruct((M,), jnp.float32),
            jax.ShapeDtypeStruct((N,), jnp.float32),
            jax.ShapeDtypeStruct((N,), jnp.float32),
        ],
    )(x_flat, embedding)

    quantized_ste = qste.reshape(x.shape)
    indices_bt = idx.reshape(B, T)
    loss = COMMITMENT_COST * (jnp.sum(lt) / jnp.sum(npad))
    avg_probs = counts / jnp.float32(N)
    perplexity = jnp.exp(-jnp.sum(avg_probs * jnp.log(avg_probs + 1e-10)))
    return (quantized_ste, loss, indices_bt, perplexity)
